# baseline (speedup 1.0000x reference)
import functools

import jax
import jax.numpy as jnp
from jax import lax
from jax.experimental import pallas as pl
from jax.experimental.pallas import tpu as pltpu
from jax.experimental.pallas import tpu_sc as plsc

N_ROWS = 1024
N_COLS = 100000
SMOOTHING = 0.1
CONFIDENCE = 1.0 - SMOOTHING

C_TC = 69760
K_SC = 30208
C_TAIL = C_TC + K_SC
TAIL_W = N_COLS - C_TAIL

R_BLK = 128
C_BLK = 13952

NC, NS, L = 2, 16, 16
ROWS_PER_CORE = N_ROWS // NC
N_CHUNKS = ROWS_PER_CORE // NS
FS = 4
RPF = NS // FS
U = 16


def _tc_kernel(x_ref, xtail_ref, t_ref, se_ref, sx_ref, xt_ref,
               s_acc, x_acc, t_acc):
    cb = pl.program_id(1)
    n_cb = pl.num_programs(1)
    t = t_ref[...]

    @pl.when(cb == 0)
    def _init():
        x = xtail_ref[...]
        lanes = jax.lax.broadcasted_iota(jnp.int32, x.shape, 1)
        cols = C_TAIL + lanes
        valid = lanes < TAIL_W
        e = jnp.exp(jnp.where(valid, jnp.minimum(x, 60.0), -jnp.inf))
        s_acc[...] = jnp.sum(e, axis=1, keepdims=True)
        x_acc[...] = jnp.sum(jnp.where(valid, x, 0.0), axis=1, keepdims=True)
        t_acc[...] = jnp.sum(jnp.where(cols == t, x, 0.0), axis=1,
                             keepdims=True)

    x = x_ref[...]
    e = jnp.exp(jnp.minimum(x, 60.0))
    s_acc[...] += jnp.sum(e, axis=1, keepdims=True)
    x_acc[...] += jnp.sum(x, axis=1, keepdims=True)
    col_ids = cb * C_BLK + jax.lax.broadcasted_iota(jnp.int32, x.shape, 1)
    t_acc[...] += jnp.sum(jnp.where(col_ids == t, x, 0.0), axis=1,
                          keepdims=True)

    @pl.when(cb == n_cb - 1)
    def _fin():
        se_ref[...] = s_acc[...]
        sx_ref[...] = x_acc[...]
        xt_ref[...] = t_acc[...]


def _tc_main(inputs, t2d):
    n_rb = N_ROWS // R_BLK
    n_cb = C_TC // C_BLK
    return pl.pallas_call(
        _tc_kernel,
        grid=(n_rb, n_cb),
        in_specs=[
            pl.BlockSpec((R_BLK, C_BLK), lambda rb, cb: (rb, cb)),
            pl.BlockSpec((R_BLK, 128), lambda rb, cb: (rb, C_TAIL // 128)),
            pl.BlockSpec((R_BLK, 1), lambda rb, cb: (rb, 0)),
        ],
        out_specs=[
            pl.BlockSpec((R_BLK, 1), lambda rb, cb: (rb, 0)),
            pl.BlockSpec((R_BLK, 1), lambda rb, cb: (rb, 0)),
            pl.BlockSpec((R_BLK, 1), lambda rb, cb: (rb, 0)),
        ],
        out_shape=[jax.ShapeDtypeStruct((N_ROWS, 1), jnp.float32),
                   jax.ShapeDtypeStruct((N_ROWS, 1), jnp.float32),
                   jax.ShapeDtypeStruct((N_ROWS, 1), jnp.float32)],
        scratch_shapes=[
            pltpu.VMEM((R_BLK, 1), jnp.float32),
            pltpu.VMEM((R_BLK, 1), jnp.float32),
            pltpu.VMEM((R_BLK, 1), jnp.float32),
        ],
    )(inputs, inputs, t2d)


def _tree(vs):
    while len(vs) > 1:
        nxt = [vs[j] + vs[j + 1] for j in range(0, len(vs) - 1, 2)]
        if len(vs) % 2:
            nxt.append(vs[-1])
        vs = nxt
    return vs[0]


_DNUMS = lax.GatherDimensionNumbers(
    offset_dims=(), collapsed_slice_dims=(0,), start_index_map=(0,))


def _sc_body(x_hbm, t_hbm, se_hbm, sx_hbm, xt_hbm,
             sp0, sp1, tbuf, tv, st_se, st_sx, st_xt, semf):
    c = lax.axis_index("c")
    s = lax.axis_index("s")
    zeros = jnp.zeros((L,), jnp.float32)
    iota = lax.broadcasted_iota(jnp.int32, (L,), 0)
    svec = jnp.full((L,), s, jnp.int32)
    sps = (sp0, sp1)

    pltpu.sync_copy(t_hbm.at[pl.ds(c * ROWS_PER_CORE, ROWS_PER_CORE)], tv)

    def fill_desc(g, spbuf):
        row0 = c * ROWS_PER_CORE + g * NS + s * RPF
        return pltpu.make_async_copy(
            x_hbm.at[pl.ds(row0, RPF), pl.ds(C_TC, K_SC)],
            spbuf.at[pl.ds(s * RPF, RPF)], semf)

    @pl.when(s < FS)
    def _prime():
        fill_desc(0, sp0).start()
        fill_desc(0, sp0).wait()

    plsc.subcore_barrier()

    for g in range(N_CHUNKS):
        cur = sps[g % 2]
        nxt = sps[(g + 1) % 2]
        if g + 1 < N_CHUNKS:
            @pl.when(s < FS)
            def _start_next(g=g, nxt=nxt):
                fill_desc(g + 1, nxt).start()

        pltpu.sync_copy(cur.at[s], tbuf)

        t16 = tv[pl.ds(g * NS, NS)]
        tloc = lax.gather(t16, svec[:, None], _DNUMS, (1,),
                          mode=lax.GatherScatterMode.PROMISE_IN_BOUNDS) - C_TC

        def ch(i, cry):
            se, sx, xt = cry
            base = i * (U * L)
            for u in range(U):
                xv = tbuf[pl.ds(base + u * L, L)]
                colv = base + u * L + iota
                if u == 0:
                    es = [jnp.exp(jnp.minimum(xv, 60.0))]
                    xs = [xv]
                    ms = [jnp.where(colv == tloc, xv, 0.0)]
                else:
                    es.append(jnp.exp(jnp.minimum(xv, 60.0)))
                    xs.append(xv)
                    ms.append(jnp.where(colv == tloc, xv, 0.0))
            return se + _tree(es), sx + _tree(xs), xt + _tree(ms)

        se16, sx16, xt16 = lax.fori_loop(0, K_SC // (U * L), ch,
                                         (zeros, zeros, zeros))
        st_se[...] = se16
        st_sx[...] = sx16
        st_xt[...] = xt16
        row = c * ROWS_PER_CORE + g * NS + s
        pltpu.sync_copy(st_se, se_hbm.at[row])
        pltpu.sync_copy(st_sx, sx_hbm.at[row])
        pltpu.sync_copy(st_xt, xt_hbm.at[row])

        if g + 1 < N_CHUNKS:
            @pl.when(s < FS)
            def _wait_next(g=g, nxt=nxt):
                fill_desc(g + 1, nxt).wait()

        plsc.subcore_barrier()


def _sc_main(inputs, t32):
    mesh = plsc.VectorSubcoreMesh(core_axis_name="c", subcore_axis_name="s")
    return pl.kernel(
        _sc_body,
        out_type=(jax.ShapeDtypeStruct((N_ROWS, L), jnp.float32),
                  jax.ShapeDtypeStruct((N_ROWS, L), jnp.float32),
                  jax.ShapeDtypeStruct((N_ROWS, L), jnp.float32)),
        mesh=mesh,
        scratch_types=[
            pltpu.MemorySpace.VMEM_SHARED((NS, K_SC), jnp.float32),
            pltpu.MemorySpace.VMEM_SHARED((NS, K_SC), jnp.float32),
            pltpu.VMEM((K_SC,), jnp.float32),
            pltpu.VMEM((ROWS_PER_CORE,), jnp.int32),
            pltpu.VMEM((L,), jnp.float32),
            pltpu.VMEM((L,), jnp.float32),
            pltpu.VMEM((L,), jnp.float32),
            pltpu.SemaphoreType.DMA,
        ],
    )(inputs, t32)


def _combine_kernel(se_tc, sx_tc, xt_tc, se_sc, sx_sc, xt_sc, out_ref):
    se = se_tc[...][:, 0] + jnp.sum(se_sc[...], axis=1)
    sx = sx_tc[...][:, 0] + jnp.sum(sx_sc[...], axis=1)
    xt = xt_tc[...][:, 0] + jnp.sum(xt_sc[...], axis=1)
    losses = (jnp.log(se) - (SMOOTHING / N_COLS) * sx - CONFIDENCE * xt)
    out_ref[...] = (jnp.sum(losses) * (1.0 / N_ROWS)).reshape(1, 1)


def _combine(se_tc, sx_tc, xt_tc, se_sc, sx_sc, xt_sc):
    return pl.pallas_call(
        _combine_kernel,
        out_shape=jax.ShapeDtypeStruct((1, 1), jnp.float32),
    )(se_tc, sx_tc, xt_tc, se_sc, sx_sc, xt_sc)


@functools.partial(jax.jit, static_argnames=())
def kernel(inputs, targets):
    t32 = targets.astype(jnp.int32)
    t2d = t32.reshape(N_ROWS, 1)
    se_tc, sx_tc, xt_tc = _tc_main(inputs, t2d)
    se_sc, sx_sc, xt_sc = _sc_main(inputs, t32)
    out = _combine(se_tc, sx_tc, xt_tc, se_sc, sx_sc, xt_sc)
    return out.reshape(())


# speedup vs baseline: 3.0340x; 3.0340x over previous
import functools

import jax
import jax.numpy as jnp
from jax.experimental import pallas as pl
from jax.experimental.pallas import tpu as pltpu

N_ROWS = 1024
N_CLS = 100000
SMOOTHING = 0.1
CONFIDENCE = 1.0 - SMOOTHING

R_CLS = 2048


def _tc_kernel(x_ref, t_ref, out_ref, s_acc, x_acc, t_acc):
    b = pl.program_id(0)
    nb = pl.num_programs(0)
    t = t_ref[...]

    @pl.when(b == 0)
    def _init():
        s_acc[...] = jnp.zeros_like(s_acc)
        x_acc[...] = jnp.zeros_like(x_acc)
        t_acc[...] = jnp.zeros_like(t_acc)

    x = x_ref[...]
    rows = b * R_CLS + jax.lax.broadcasted_iota(jnp.int32, x.shape, 0)
    match = rows == t

    @pl.when(b != nb - 1)
    def _full():
        e = jnp.exp(jnp.minimum(x, 60.0))
        s_acc[...] += jnp.sum(e, axis=0, keepdims=True)
        x_acc[...] += jnp.sum(x, axis=0, keepdims=True)
        t_acc[...] += jnp.sum(jnp.where(match, x, 0.0), axis=0, keepdims=True)

    @pl.when(b == nb - 1)
    def _masked():
        valid = rows < N_CLS
        e = jnp.exp(jnp.where(valid, jnp.minimum(x, 60.0), -jnp.inf))
        se = s_acc[...] + jnp.sum(e, axis=0, keepdims=True)
        sx = x_acc[...] + jnp.sum(jnp.where(valid, x, 0.0), axis=0,
                                  keepdims=True)
        xt = t_acc[...] + jnp.sum(jnp.where(match, x, 0.0), axis=0,
                                  keepdims=True)
        losses = (jnp.log(se) - (SMOOTHING / N_CLS) * sx - CONFIDENCE * xt)
        out_ref[...] = (jnp.sum(losses) * (1.0 / N_ROWS)).reshape(1, 1)


@functools.partial(jax.jit, static_argnames=())
def kernel(inputs, targets):
    xt = inputs.T
    t2d = targets.astype(jnp.int32).reshape(1, N_ROWS)
    nb = pl.cdiv(N_CLS, R_CLS)

    out = pl.pallas_call(
        _tc_kernel,
        grid=(nb,),
        in_specs=[
            pl.BlockSpec((R_CLS, N_ROWS), lambda b: (b, 0)),
            pl.BlockSpec((1, N_ROWS), lambda b: (0, 0)),
        ],
        out_specs=pl.BlockSpec((1, 1), lambda b: (0, 0)),
        out_shape=jax.ShapeDtypeStruct((1, 1), jnp.float32),
        scratch_shapes=[
            pltpu.VMEM((1, N_ROWS), jnp.float32),
            pltpu.VMEM((1, N_ROWS), jnp.float32),
            pltpu.VMEM((1, N_ROWS), jnp.float32),
        ],
    )(xt, t2d)
    return out.reshape(())


# speedup vs baseline: 3.3274x; 1.0967x over previous
import functools

import jax
import jax.numpy as jnp
from jax import lax
from jax.experimental import pallas as pl
from jax.experimental.pallas import tpu as pltpu
from jax.experimental.pallas import tpu_sc as plsc

N_ROWS = 1024
N_CLS = 100000
SMOOTHING = 0.1
CONFIDENCE = 1.0 - SMOOTHING

R_CLS = 2048
NB_TC = 39
C0_SC = NB_TC * R_CLS
S_SC = N_CLS - C0_SC
NC, NS, L = 2, 16, 16
S_PC = S_SC // NC
CC = 592
N_CHUNKS = S_PC // CC
CH = CC // 2
FS = 2
CPF = CC // FS
NV = N_ROWS // (8 * L)


def _tc_kernel(x_ref, t_ref, se_ref, sx_ref, xt_ref, s_acc, x_acc, t_acc):
    b = pl.program_id(0)
    nb = pl.num_programs(0)
    t = t_ref[...]

    @pl.when(b == 0)
    def _init():
        s_acc[...] = jnp.zeros_like(s_acc)
        x_acc[...] = jnp.zeros_like(x_acc)
        t_acc[...] = jnp.zeros_like(t_acc)

    x = x_ref[...]
    rows = b * R_CLS + jax.lax.broadcasted_iota(jnp.int32, x.shape, 0)
    e = jnp.exp(jnp.minimum(x, 60.0))
    s_acc[...] += jnp.sum(e, axis=0, keepdims=True)
    x_acc[...] += jnp.sum(x, axis=0, keepdims=True)
    t_acc[...] += jnp.sum(jnp.where(rows == t, x, 0.0), axis=0, keepdims=True)

    @pl.when(b == nb - 1)
    def _fin():
        se_ref[...] = s_acc[...]
        sx_ref[...] = x_acc[...]
        xt_ref[...] = t_acc[...]


def _tc_main(xT, t2d):
    return pl.pallas_call(
        _tc_kernel,
        grid=(NB_TC,),
        in_specs=[
            pl.BlockSpec((R_CLS, N_ROWS), lambda b: (b, 0)),
            pl.BlockSpec((1, N_ROWS), lambda b: (0, 0)),
        ],
        out_specs=[
            pl.BlockSpec((1, N_ROWS), lambda b: (0, 0)),
            pl.BlockSpec((1, N_ROWS), lambda b: (0, 0)),
            pl.BlockSpec((1, N_ROWS), lambda b: (0, 0)),
        ],
        out_shape=[jax.ShapeDtypeStruct((1, N_ROWS), jnp.float32)] * 3,
        scratch_shapes=[pltpu.VMEM((1, N_ROWS), jnp.float32)] * 3,
    )(xT, t2d)


def _sc_body(x_hbm, t_hbm, se_hbm, sx_hbm, xt_hbm,
             sp0, sp1, tbuf, tv, st, semf):
    c = lax.axis_index("c")
    s = lax.axis_index("s")
    h = s // 8
    p = s % 8
    sps = (sp0, sp1)
    zeros = jnp.zeros((L,), jnp.float32)

    pltpu.sync_copy(t_hbm.at[0], tv)
    tks = [tv[pl.ds(p * 128 + k * L, L)] for k in range(8)]

    def fill_desc(g, spbuf):
        cls0 = C0_SC + c * S_PC + g * CC + s * CPF
        return pltpu.make_async_copy(
            x_hbm.at[pl.ds(cls0, CPF)],
            spbuf.at[pl.ds(s * CPF, CPF)], semf)

    @pl.when(s < FS)
    def _prime():
        fill_desc(0, sp0).start()
        fill_desc(0, sp0).wait()

    plsc.subcore_barrier()

    accs = [zeros] * 24

    for g in range(N_CHUNKS):
        cur = sps[g % 2]
        nxt = sps[(g + 1) % 2]
        if g + 1 < N_CHUNKS:
            @pl.when(s < FS)
            def _start_next(g=g, nxt=nxt):
                fill_desc(g + 1, nxt).start()

        pltpu.sync_copy(cur.at[pl.ds(h * CH, CH), pl.ds(p * 128, 128)], tbuf)

        cls_base = C0_SC + c * S_PC + g * CC + h * CH

        def cls_body(i, cry, cls_base=cls_base):
            cry = list(cry)
            clsg = cls_base + i
            for k in range(8):
                xv = tbuf[i, pl.ds(k * L, L)]
                cry[k] = cry[k] + jnp.exp(jnp.minimum(xv, 60.0))
                cry[8 + k] = cry[8 + k] + xv
                cry[16 + k] = cry[16 + k] + jnp.where(tks[k] == clsg, xv, 0.0)
            return tuple(cry)

        accs = list(lax.fori_loop(0, CH, cls_body, tuple(accs)))

        if g + 1 < N_CHUNKS:
            @pl.when(s < FS)
            def _wait_next(g=g, nxt=nxt):
                fill_desc(g + 1, nxt).wait()

        plsc.subcore_barrier()

    out_row = c * 2 + h
    for name, ref, off in ((0, se_hbm, 0), (1, sx_hbm, 8), (2, xt_hbm, 16)):
        for k in range(8):
            st[pl.ds(k * L, L)] = accs[off + k]
        pltpu.sync_copy(st, ref.at[out_row, pl.ds(p * 128, 128)])


def _sc_main(xT, t2d):
    mesh = plsc.VectorSubcoreMesh(core_axis_name="c", subcore_axis_name="s")
    return pl.kernel(
        _sc_body,
        out_type=(jax.ShapeDtypeStruct((4, N_ROWS), jnp.float32),
                  jax.ShapeDtypeStruct((4, N_ROWS), jnp.float32),
                  jax.ShapeDtypeStruct((4, N_ROWS), jnp.float32)),
        mesh=mesh,
        scratch_types=[
            pltpu.MemorySpace.VMEM_SHARED((CC, N_ROWS), jnp.float32),
            pltpu.MemorySpace.VMEM_SHARED((CC, N_ROWS), jnp.float32),
            pltpu.VMEM((CH, 128), jnp.float32),
            pltpu.VMEM((N_ROWS,), jnp.int32),
            pltpu.VMEM((128,), jnp.float32),
            pltpu.SemaphoreType.DMA,
        ],
    )(xT, t2d)


def _combine_kernel(se_tc, sx_tc, xt_tc, se_sc, sx_sc, xt_sc, out_ref):
    se = se_tc[...][0] + jnp.sum(se_sc[...], axis=0)
    sx = sx_tc[...][0] + jnp.sum(sx_sc[...], axis=0)
    xt = xt_tc[...][0] + jnp.sum(xt_sc[...], axis=0)
    losses = (jnp.log(se) - (SMOOTHING / N_CLS) * sx - CONFIDENCE * xt)
    out_ref[...] = (jnp.sum(losses) * (1.0 / N_ROWS)).reshape(1, 1)


@functools.partial(jax.jit, static_argnames=())
def kernel(inputs, targets):
    xT = inputs.T
    t2d = targets.astype(jnp.int32).reshape(1, N_ROWS)
    se_sc, sx_sc, xt_sc = _sc_main(xT, t2d)
    se_tc, sx_tc, xt_tc = _tc_main(xT, t2d)
    out = pl.pallas_call(
        _combine_kernel,
        out_shape=jax.ShapeDtypeStruct((1, 1), jnp.float32),
    )(se_tc, sx_tc, xt_tc, se_sc, sx_sc, xt_sc)
    return out.reshape(())
